# Initial kernel scaffold; baseline (speedup 1.0000x reference)
import functools

import jax
import jax.numpy as jnp
from jax.experimental import pallas as pl
from jax.experimental.pallas import tpu as pltpu
from jax.experimental.pallas import tpu_sc as plsc

N_SUBCORES = 16
N_CORES = 2
N_WORKERS = N_CORES * N_SUBCORES
GATHER_W = 400
SCATTER_W = 400


def _sc_mesh():
    return plsc.VectorSubcoreMesh(core_axis_name="c", subcore_axis_name="s")


def _gather_sum(a, b, src, dst):
    e_total = src.shape[0]
    d = a.shape[1]
    per_worker = e_total // N_WORKERS
    w = min(GATHER_W, per_worker)

    @pl.kernel(
        out_type=jax.ShapeDtypeStruct((e_total, d), jnp.float32),
        mesh=_sc_mesh(),
        scratch_types=[
            pltpu.VMEM((w,), jnp.int32),
            pltpu.VMEM((w,), jnp.int32),
            pltpu.VMEM((w, d), jnp.float32),
            pltpu.SemaphoreType.DMA,
            pltpu.SemaphoreType.DMA,
        ],
    )
    def k(a_hbm, b_hbm, src_hbm, dst_hbm, out_hbm, idx_s, idx_d, rows, sem1,
          sem2):
        c = jax.lax.axis_index("c")
        s = jax.lax.axis_index("s")
        wid = c * N_SUBCORES + s
        base0 = wid * per_worker

        @pl.loop(0, per_worker, step=w)
        def _(off):
            base = base0 + off
            cp1 = pltpu.async_copy(src_hbm.at[pl.ds(base, w)], idx_s, sem1)
            cp2 = pltpu.async_copy(dst_hbm.at[pl.ds(base, w)], idx_d, sem2)
            cp1.wait()
            cp2.wait()
            pltpu.async_copy(a_hbm.at[idx_s], rows, sem1).wait()
            pltpu.async_copy(b_hbm.at[idx_d], rows, sem1, add=True).wait()
            pltpu.async_copy(rows, out_hbm.at[pl.ds(base, w)], sem1).wait()

    return k(a, b, src, dst)


def _segment_sum(vals, dst, num_nodes):
    e_total, d = vals.shape
    half = e_total // N_CORES
    per_worker = half // N_SUBCORES
    w = min(SCATTER_W, per_worker)
    rows_per_sub = num_nodes // N_SUBCORES
    zeros = jnp.zeros((rows_per_sub, d), jnp.float32)

    @pl.kernel(
        out_type=jax.ShapeDtypeStruct((N_CORES, num_nodes, d), jnp.float32),
        mesh=_sc_mesh(),
        scratch_types=[
            pltpu.VMEM((w,), jnp.int32),
            pltpu.VMEM((w, d), jnp.float32),
            pltpu.VMEM_SHARED((num_nodes, d), jnp.float32),
            pltpu.SemaphoreType.DMA,
            pltpu.SemaphoreType.DMA,
        ],
    )
    def k(v_hbm, dst_hbm, zero_hbm, out_hbm, idx_d, rows, acc, sem1, sem2):
        c = jax.lax.axis_index("c")
        s = jax.lax.axis_index("s")
        my_rows = pl.ds(s * rows_per_sub, rows_per_sub)
        pltpu.async_copy(zero_hbm, acc.at[my_rows], sem1).wait()
        plsc.subcore_barrier()
        base0 = c * half + s * per_worker

        @pl.loop(0, per_worker, step=w)
        def _(off):
            base = base0 + off
            cp1 = pltpu.async_copy(dst_hbm.at[pl.ds(base, w)], idx_d, sem1)
            cp2 = pltpu.async_copy(v_hbm.at[pl.ds(base, w)], rows, sem2)
            cp1.wait()
            cp2.wait()
            pltpu.async_copy(rows, acc.at[idx_d], sem1, add=True).wait()

        plsc.subcore_barrier()
        pltpu.async_copy(acc.at[my_rows], out_hbm.at[c, my_rows], sem1).wait()

    return k(vals, dst, zeros)


_NODE_BLK = 2000
_EDGE_BLK = 2000


def _prep_body(x_ref, wa_ref, wb_ref, b_ref, a_ref, bd_ref):
    x = x_ref[...]
    a_ref[...] = (
        jnp.dot(x, wa_ref[...], preferred_element_type=jnp.float32) + b_ref[...]
    )
    bd_ref[...] = jnp.dot(x, wb_ref[...], preferred_element_type=jnp.float32)


def _prep(x, wa, wb, b1):
    n, d = x.shape
    grid = n // _NODE_BLK
    return pl.pallas_call(
        _prep_body,
        grid=(grid,),
        in_specs=[
            pl.BlockSpec((_NODE_BLK, d), lambda i: (i, 0)),
            pl.BlockSpec(wa.shape, lambda i: (0, 0)),
            pl.BlockSpec(wb.shape, lambda i: (0, 0)),
            pl.BlockSpec((1, b1.shape[1]), lambda i: (0, 0)),
        ],
        out_specs=[
            pl.BlockSpec((_NODE_BLK, wa.shape[1]), lambda i: (i, 0)),
            pl.BlockSpec((_NODE_BLK, wb.shape[1]), lambda i: (i, 0)),
        ],
        out_shape=[
            jax.ShapeDtypeStruct((n, wa.shape[1]), jnp.float32),
            jax.ShapeDtypeStruct((n, wb.shape[1]), jnp.float32),
        ],
    )(x, wa, wb, b1)


def _edge_mlp_body(s_ref, e_ref, wc_ref, w2_ref, b2_ref, out_ref):
    h = s_ref[...] + jnp.dot(
        e_ref[...], wc_ref[...], preferred_element_type=jnp.float32
    )
    h = jnp.maximum(h, 0.0)
    out_ref[...] = (
        jnp.dot(h, w2_ref[...], preferred_element_type=jnp.float32) + b2_ref[...]
    )


def _edge_mlp(s, e, wc, w2, b2):
    e_total, hid = s.shape
    dout = w2.shape[1]
    grid = e_total // _EDGE_BLK
    return pl.pallas_call(
        _edge_mlp_body,
        grid=(grid,),
        in_specs=[
            pl.BlockSpec((_EDGE_BLK, hid), lambda i: (i, 0)),
            pl.BlockSpec((_EDGE_BLK, e.shape[1]), lambda i: (i, 0)),
            pl.BlockSpec(wc.shape, lambda i: (0, 0)),
            pl.BlockSpec(w2.shape, lambda i: (0, 0)),
            pl.BlockSpec((1, dout), lambda i: (0, 0)),
        ],
        out_specs=pl.BlockSpec((_EDGE_BLK, dout), lambda i: (i, 0)),
        out_shape=jax.ShapeDtypeStruct((e_total, dout), jnp.float32),
    )(s, e, wc, w2, b2)


def _edge_mlp3_body(s_ref, e1_ref, e2_ref, wc_ref, w2_ref, b2_ref, pad_ref,
                    col_ref):
    e_sum = e1_ref[...] + e2_ref[...]
    h = s_ref[...] + jnp.dot(
        e_sum, wc_ref[...], preferred_element_type=jnp.float32
    )
    h = jnp.maximum(h, 0.0)
    out = jnp.dot(h, w2_ref[...], preferred_element_type=jnp.float32) + b2_ref[...]
    pad_ref[...] = out
    col_ref[...] = out[:, 0:1]


def _edge_mlp3(s, e1, e2, wc, w2p, b2p):
    e_total, hid = s.shape
    dpad = w2p.shape[1]
    grid = e_total // _EDGE_BLK
    return pl.pallas_call(
        _edge_mlp3_body,
        grid=(grid,),
        in_specs=[
            pl.BlockSpec((_EDGE_BLK, hid), lambda i: (i, 0)),
            pl.BlockSpec((_EDGE_BLK, e1.shape[1]), lambda i: (i, 0)),
            pl.BlockSpec((_EDGE_BLK, e2.shape[1]), lambda i: (i, 0)),
            pl.BlockSpec(wc.shape, lambda i: (0, 0)),
            pl.BlockSpec(w2p.shape, lambda i: (0, 0)),
            pl.BlockSpec((1, dpad), lambda i: (0, 0)),
        ],
        out_specs=[
            pl.BlockSpec((_EDGE_BLK, dpad), lambda i: (i, 0)),
            pl.BlockSpec((_EDGE_BLK, 1), lambda i: (i, 0)),
        ],
        out_shape=[
            jax.ShapeDtypeStruct((e_total, dpad), jnp.float32),
            jax.ShapeDtypeStruct((e_total, 1), jnp.float32),
        ],
    )(s, e1, e2, wc, w2p, b2p)


def _node_mlp_body(x_ref, p0_ref, p1_ref, w1a_ref, w1b_ref, b1_ref, w2_ref,
                   b2_ref, out_ref, *, agg_is_col, residual):
    x = x_ref[...]
    agg = p0_ref[...] + p1_ref[...]
    if agg_is_col:
        term = agg[:, 0:1] * w1b_ref[...]
    else:
        term = jnp.dot(agg, w1b_ref[...], preferred_element_type=jnp.float32)
    h = (
        jnp.dot(x, w1a_ref[...], preferred_element_type=jnp.float32)
        + term
        + b1_ref[...]
    )
    h = jnp.maximum(h, 0.0)
    out = jnp.dot(h, w2_ref[...], preferred_element_type=jnp.float32) + b2_ref[...]
    if residual:
        out = out + x
    out_ref[...] = out


def _node_mlp(x, p0, p1, w1a, w1b, b1, w2, b2, *, agg_is_col, residual):
    n, d = x.shape
    dagg = p0.shape[1]
    dout = w2.shape[1]
    grid = n // _NODE_BLK
    body = functools.partial(
        _node_mlp_body, agg_is_col=agg_is_col, residual=residual
    )
    return pl.pallas_call(
        body,
        grid=(grid,),
        in_specs=[
            pl.BlockSpec((_NODE_BLK, d), lambda i: (i, 0)),
            pl.BlockSpec((_NODE_BLK, dagg), lambda i: (i, 0)),
            pl.BlockSpec((_NODE_BLK, dagg), lambda i: (i, 0)),
            pl.BlockSpec(w1a.shape, lambda i: (0, 0)),
            pl.BlockSpec(w1b.shape, lambda i: (0, 0)),
            pl.BlockSpec((1, b1.shape[1]), lambda i: (0, 0)),
            pl.BlockSpec(w2.shape, lambda i: (0, 0)),
            pl.BlockSpec((1, dout), lambda i: (0, 0)),
        ],
        out_specs=pl.BlockSpec((_NODE_BLK, dout), lambda i: (i, 0)),
        out_shape=jax.ShapeDtypeStruct((n, dout), jnp.float32),
    )(x, p0, p1, w1a, w1b, b1, w2, b2)


def kernel(x, edge_index, edge_attr, params):
    n = x.shape[0]
    src = edge_index[0]
    dst = edge_index[1]

    def prep_and_gather(p, node_feat):
        ew1 = p["eW1"]
        dn = node_feat.shape[1]
        a, bd = _prep(node_feat, ew1[:dn], ew1[dn : 2 * dn],
                      p["eb1"].reshape(1, -1))
        return _gather_sum(a, bd, src, dst)

    p = params[0]
    dn = x.shape[1]
    s1 = prep_and_gather(p, x)
    en1 = _edge_mlp(s1, edge_attr, p["eW1"][2 * dn :], p["eW2"],
                    p["eb2"].reshape(1, -1))
    part1 = _segment_sum(en1, dst, n)
    nw1 = p["nW1"]
    x1 = _node_mlp(
        x, part1[0], part1[1], nw1[:dn], nw1[dn:], p["nb1"].reshape(1, -1),
        p["nW2"], p["nb2"].reshape(1, -1), agg_is_col=False, residual=False,
    )

    p = params[1]
    dn = x1.shape[1]
    s2 = prep_and_gather(p, x1)
    en2 = _edge_mlp(s2, en1, p["eW1"][2 * dn :], p["eW2"],
                    p["eb2"].reshape(1, -1))
    part2 = _segment_sum(en2, dst, n)
    nw1 = p["nW1"]
    x2 = _node_mlp(
        x1, part2[0], part2[1], nw1[:dn], nw1[dn:], p["nb1"].reshape(1, -1),
        p["nW2"], p["nb2"].reshape(1, -1), agg_is_col=False, residual=True,
    )

    p = params[2]
    dn = x2.shape[1]
    s3 = prep_and_gather(p, x2)
    w2p = jnp.concatenate(
        [p["eW2"], jnp.zeros((p["eW2"].shape[0], 15), jnp.float32)], axis=1
    )
    b2p = jnp.concatenate(
        [p["eb2"], jnp.zeros((15,), jnp.float32)]
    ).reshape(1, -1)
    en3_pad, en3 = _edge_mlp3(s3, en1, en2, p["eW1"][2 * dn :], w2p, b2p)
    part3 = _segment_sum(en3_pad, dst, n)
    nw1 = p["nW1"]
    x3 = _node_mlp(
        x2, part3[0], part3[1], nw1[:dn], nw1[dn:], p["nb1"].reshape(1, -1),
        p["nW2"], p["nb2"].reshape(1, -1), agg_is_col=True, residual=False,
    )
    return (x3, en3)


# speedup vs baseline: 3.0098x; 3.0098x over previous
import functools

import jax
import jax.numpy as jnp
from jax.experimental import pallas as pl
from jax.experimental.pallas import tpu as pltpu
from jax.experimental.pallas import tpu_sc as plsc

N_SUBCORES = 16
N_CORES = 2
N_WORKERS = N_CORES * N_SUBCORES
GATHER_W = 400
SCATTER_W = 400
NODE_CHUNK = 200


def _sc_mesh():
    return plsc.VectorSubcoreMesh(core_axis_name="c", subcore_axis_name="s")


def _spmem_chunked(s, num_rows, fn):
    for j in range(num_rows // NODE_CHUNK):
        def chunk_fn(j=j):
            fn(j)

        pl.when(s == j % N_SUBCORES)(chunk_fn)


def _gather_sum(a, b, src, dst):
    e_total = src.shape[0]
    d = a.shape[1]
    per_worker = e_total // N_WORKERS
    w = min(GATHER_W, per_worker)

    @pl.kernel(
        out_type=jax.ShapeDtypeStruct((e_total, d), jnp.float32),
        mesh=_sc_mesh(),
        scratch_types=[
            pltpu.VMEM((w,), jnp.int32),
            pltpu.VMEM((w,), jnp.int32),
            pltpu.VMEM((w, d), jnp.float32),
            pltpu.SemaphoreType.DMA,
            pltpu.SemaphoreType.DMA,
        ],
    )
    def k(a_hbm, b_hbm, src_hbm, dst_hbm, out_hbm, idx_s, idx_d, rows, sem1,
          sem2):
        c = jax.lax.axis_index("c")
        s = jax.lax.axis_index("s")
        wid = c * N_SUBCORES + s
        base0 = wid * per_worker

        @pl.loop(0, per_worker, step=w)
        def _(off):
            base = base0 + off
            cp1 = pltpu.async_copy(src_hbm.at[pl.ds(base, w)], idx_s, sem1)
            cp2 = pltpu.async_copy(dst_hbm.at[pl.ds(base, w)], idx_d, sem2)
            cp1.wait()
            cp2.wait()
            pltpu.async_copy(a_hbm.at[idx_s], rows, sem1).wait()
            pltpu.async_copy(b_hbm.at[idx_d], rows, sem1, add=True).wait()
            pltpu.async_copy(rows, out_hbm.at[pl.ds(base, w)], sem1).wait()

    return k(a, b, src, dst)


def _segment_sum(vals, dst, num_nodes):
    e_total, d = vals.shape
    half_nodes = num_nodes // N_CORES
    per_worker = e_total // N_SUBCORES
    w = min(SCATTER_W, per_worker)
    zeros = jnp.zeros((NODE_CHUNK, d), jnp.float32)
    iota = jnp.arange(half_nodes, dtype=jnp.int32)
    dstc = jnp.concatenate([
        jnp.where(dst < half_nodes, dst, num_nodes),
        jnp.where(dst >= half_nodes, dst - half_nodes, num_nodes),
    ])

    @pl.kernel(
        out_type=jax.ShapeDtypeStruct((N_CORES, half_nodes, d), jnp.float32),
        mesh=_sc_mesh(),
        scratch_types=[
            pltpu.VMEM((w,), jnp.int32),
            pltpu.VMEM((NODE_CHUNK,), jnp.int32),
            pltpu.VMEM((w, d), jnp.float32),
            pltpu.VMEM_SHARED((half_nodes, d), jnp.float32),
            pltpu.SemaphoreType.DMA,
            pltpu.SemaphoreType.DMA,
        ],
    )
    def k(v_hbm, dstc_hbm, zero_hbm, iota_hbm, out_hbm, idx_d,
          idx_c, rows, acc, sem1, sem2):
        c = jax.lax.axis_index("c")
        s = jax.lax.axis_index("s")

        pltpu.async_copy(zero_hbm, rows.at[pl.ds(0, NODE_CHUNK)], sem1).wait()

        def zero_chunk(j):
            pltpu.async_copy(
                iota_hbm.at[pl.ds(j * NODE_CHUNK, NODE_CHUNK)], idx_c,
                sem1).wait()
            pltpu.async_copy(rows.at[pl.ds(0, NODE_CHUNK)], acc.at[idx_c],
                             sem1).wait()

        _spmem_chunked(s, half_nodes, zero_chunk)

        plsc.subcore_barrier()
        base0 = s * per_worker

        @pl.loop(0, per_worker, step=w)
        def _(off):
            base = base0 + off
            cp1 = pltpu.async_copy(
                dstc_hbm.at[pl.ds(c * e_total + base, w)], idx_d, sem1)
            cp2 = pltpu.async_copy(v_hbm.at[pl.ds(base, w)], rows, sem2)
            cp1.wait()
            cp2.wait()
            pltpu.async_copy(
                rows,
                acc.at[plsc.Indices(idx_d, ignored_value=num_nodes)],
                sem1, add=True,
            ).wait()

        plsc.subcore_barrier()

        def drain(j):
            blk = pl.ds(j * NODE_CHUNK, NODE_CHUNK)
            buf = rows.at[pl.ds(0, NODE_CHUNK)]
            pltpu.async_copy(iota_hbm.at[blk], idx_c, sem1).wait()
            pltpu.async_copy(acc.at[idx_c], buf, sem1).wait()
            pltpu.async_copy(buf, out_hbm.at[c, blk], sem1).wait()

        _spmem_chunked(s, half_nodes, drain)

    out = k(vals, dstc, zeros, iota)
    return out.reshape(num_nodes, d)


_NODE_BLK = 2000
_EDGE_BLK = 2000


def _prep_body(x_ref, wa_ref, wb_ref, b_ref, a_ref, bd_ref):
    x = x_ref[...]
    a_ref[...] = (
        jnp.dot(x, wa_ref[...], preferred_element_type=jnp.float32) + b_ref[...]
    )
    bd_ref[...] = jnp.dot(x, wb_ref[...], preferred_element_type=jnp.float32)


def _prep(x, wa, wb, b1):
    n, d = x.shape
    grid = n // _NODE_BLK
    return pl.pallas_call(
        _prep_body,
        grid=(grid,),
        in_specs=[
            pl.BlockSpec((_NODE_BLK, d), lambda i: (i, 0)),
            pl.BlockSpec(wa.shape, lambda i: (0, 0)),
            pl.BlockSpec(wb.shape, lambda i: (0, 0)),
            pl.BlockSpec((1, b1.shape[1]), lambda i: (0, 0)),
        ],
        out_specs=[
            pl.BlockSpec((_NODE_BLK, wa.shape[1]), lambda i: (i, 0)),
            pl.BlockSpec((_NODE_BLK, wb.shape[1]), lambda i: (i, 0)),
        ],
        out_shape=[
            jax.ShapeDtypeStruct((n, wa.shape[1]), jnp.float32),
            jax.ShapeDtypeStruct((n, wb.shape[1]), jnp.float32),
        ],
    )(x, wa, wb, b1)


def _edge_mlp_body(s_ref, *refs, n_e):
    e_refs = refs[:n_e]
    wc_refs = refs[n_e : 2 * n_e]
    w2_ref, b2_ref, out_ref = refs[2 * n_e :]
    h = s_ref[...]
    for e_ref, wc_ref in zip(e_refs, wc_refs):
        h = h + jnp.dot(
            e_ref[...], wc_ref[...], preferred_element_type=jnp.float32
        )
    h = jnp.maximum(h, 0.0)
    out_ref[...] = (
        jnp.dot(h, w2_ref[...], preferred_element_type=jnp.float32) + b2_ref[...]
    )


def _edge_mlp(s, e_list, wc_list, w2, b2):
    e_total, hid = s.shape
    dout = w2.shape[1]
    grid = e_total // _EDGE_BLK
    n_e = len(e_list)
    body = functools.partial(_edge_mlp_body, n_e=n_e)
    e_specs = [
        pl.BlockSpec((_EDGE_BLK, e.shape[1]), lambda i: (i, 0)) for e in e_list
    ]
    wc_specs = [pl.BlockSpec(wc.shape, lambda i: (0, 0)) for wc in wc_list]
    return pl.pallas_call(
        body,
        grid=(grid,),
        in_specs=[
            pl.BlockSpec((_EDGE_BLK, hid), lambda i: (i, 0)),
            *e_specs,
            *wc_specs,
            pl.BlockSpec(w2.shape, lambda i: (0, 0)),
            pl.BlockSpec((1, dout), lambda i: (0, 0)),
        ],
        out_specs=pl.BlockSpec((_EDGE_BLK, dout), lambda i: (i, 0)),
        out_shape=jax.ShapeDtypeStruct((e_total, dout), jnp.float32),
    )(s, *e_list, *wc_list, w2, b2)


def _edge_mlp3_body(s_ref, e1_ref, e2_ref, wc_ref, w2_ref, b2_ref, pad_ref,
                    col_ref):
    h = s_ref[...] + jnp.dot(
        e1_ref[...] + e2_ref[...], wc_ref[...],
        preferred_element_type=jnp.float32,
    )
    h = jnp.maximum(h, 0.0)
    out = jnp.dot(h, w2_ref[...], preferred_element_type=jnp.float32) + b2_ref[...]
    pad_ref[...] = out
    col_ref[...] = out[:, 0:1]


def _edge_mlp3(s, e1, e2, wc, w2p, b2p):
    e_total, hid = s.shape
    dpad = w2p.shape[1]
    grid = e_total // _EDGE_BLK
    return pl.pallas_call(
        _edge_mlp3_body,
        grid=(grid,),
        in_specs=[
            pl.BlockSpec((_EDGE_BLK, hid), lambda i: (i, 0)),
            pl.BlockSpec((_EDGE_BLK, e1.shape[1]), lambda i: (i, 0)),
            pl.BlockSpec((_EDGE_BLK, e2.shape[1]), lambda i: (i, 0)),
            pl.BlockSpec(wc.shape, lambda i: (0, 0)),
            pl.BlockSpec(w2p.shape, lambda i: (0, 0)),
            pl.BlockSpec((1, dpad), lambda i: (0, 0)),
        ],
        out_specs=[
            pl.BlockSpec((_EDGE_BLK, dpad), lambda i: (i, 0)),
            pl.BlockSpec((_EDGE_BLK, 1), lambda i: (i, 0)),
        ],
        out_shape=[
            jax.ShapeDtypeStruct((e_total, dpad), jnp.float32),
            jax.ShapeDtypeStruct((e_total, 1), jnp.float32),
        ],
    )(s, e1, e2, wc, w2p, b2p)


def _node_mlp_body(x_ref, agg_ref, w1a_ref, w1b_ref, b1_ref, w2_ref,
                   b2_ref, out_ref, *, agg_is_col, residual):
    x = x_ref[...]
    if agg_is_col:
        term = agg_ref[...][:, 0:1] * w1b_ref[...]
    else:
        term = jnp.dot(
            agg_ref[...], w1b_ref[...], preferred_element_type=jnp.float32
        )
    h = (
        jnp.dot(x, w1a_ref[...], preferred_element_type=jnp.float32)
        + term
        + b1_ref[...]
    )
    h = jnp.maximum(h, 0.0)
    out = jnp.dot(h, w2_ref[...], preferred_element_type=jnp.float32) + b2_ref[...]
    if residual:
        out = out + x
    out_ref[...] = out


def _node_mlp(x, agg, w1a, w1b, b1, w2, b2, *, agg_is_col, residual):
    n, d = x.shape
    dagg = agg.shape[1]
    dout = w2.shape[1]
    grid = n // _NODE_BLK
    body = functools.partial(
        _node_mlp_body, agg_is_col=agg_is_col, residual=residual
    )
    return pl.pallas_call(
        body,
        grid=(grid,),
        in_specs=[
            pl.BlockSpec((_NODE_BLK, d), lambda i: (i, 0)),
            pl.BlockSpec((_NODE_BLK, dagg), lambda i: (i, 0)),
            pl.BlockSpec(w1a.shape, lambda i: (0, 0)),
            pl.BlockSpec(w1b.shape, lambda i: (0, 0)),
            pl.BlockSpec((1, b1.shape[1]), lambda i: (0, 0)),
            pl.BlockSpec(w2.shape, lambda i: (0, 0)),
            pl.BlockSpec((1, dout), lambda i: (0, 0)),
        ],
        out_specs=pl.BlockSpec((_NODE_BLK, dout), lambda i: (i, 0)),
        out_shape=jax.ShapeDtypeStruct((n, dout), jnp.float32),
    )(x, agg, w1a, w1b, b1, w2, b2)


def kernel(x, edge_index, edge_attr, params):
    n = x.shape[0]
    src = edge_index[0]
    dst = edge_index[1]

    def prep_and_gather(p, node_feat):
        ew1 = p["eW1"]
        dn = node_feat.shape[1]
        a, bd = _prep(node_feat, ew1[:dn], ew1[dn : 2 * dn],
                      p["eb1"].reshape(1, -1))
        return _gather_sum(a, bd, src, dst)

    def node_update(p, node_feat, agg, *, agg_is_col, residual):
        dn = node_feat.shape[1]
        nw1 = p["nW1"]
        return _node_mlp(
            node_feat, agg, nw1[:dn], nw1[dn:], p["nb1"].reshape(1, -1),
            p["nW2"], p["nb2"].reshape(1, -1),
            agg_is_col=agg_is_col, residual=residual,
        )

    p = params[0]
    dn = x.shape[1]
    s1 = prep_and_gather(p, x)
    en1 = _edge_mlp(s1, [edge_attr], [p["eW1"][2 * dn :]], p["eW2"],
                    p["eb2"].reshape(1, -1))
    agg1 = _segment_sum(en1, dst, n)
    x1 = node_update(p, x, agg1, agg_is_col=False, residual=False)

    p = params[1]
    dn = x1.shape[1]
    s2 = prep_and_gather(p, x1)
    en2 = _edge_mlp(s2, [en1], [p["eW1"][2 * dn :]], p["eW2"],
                    p["eb2"].reshape(1, -1))
    agg2 = _segment_sum(en2, dst, n)
    x2 = node_update(p, x1, agg2, agg_is_col=False, residual=True)

    p = params[2]
    dn = x2.shape[1]
    s3 = prep_and_gather(p, x2)
    w2p = jnp.concatenate(
        [p["eW2"], jnp.zeros((p["eW2"].shape[0], 127), jnp.float32)], axis=1
    )
    b2p = jnp.concatenate(
        [p["eb2"], jnp.zeros((127,), jnp.float32)]
    ).reshape(1, -1)
    en3_pad, en3 = _edge_mlp3(s3, en1, en2, p["eW1"][2 * dn :], w2p, b2p)
    agg3 = _segment_sum(en3_pad, dst, n)
    x3 = node_update(p, x2, agg3, agg_is_col=True, residual=False)
    return (x3, en3)
